# Initial kernel scaffold; baseline (speedup 1.0000x reference)
import jax
import jax.numpy as jnp
from jax.experimental import pallas as pl


def kernel(nfeats, efeats, edge_index, W_fc, W_edge, b_edge, W_coef):
    raise NotImplementedError("write your pallas kernel here")


# speedup vs baseline: 3.6810x; 3.6810x over previous
import functools

import jax
import jax.numpy as jnp
from jax import lax
from jax.experimental import pallas as pl
from jax.experimental.pallas import tpu as pltpu
from jax.experimental.pallas import tpu_sc as plsc

NC = 2
NS = 16
NW = NC * NS
LANES = 16

_NEG_BIG = -3.0e38


def _leaky(x):
    return jnp.where(x >= 0, x, 0.01 * x)


def _node_proj_body(x_ref, wfc_ref, wa_ref, wc_ref, b_ref, z_ref, zsb_ref, zd_ref):
    x = x_ref[...]
    z = jnp.dot(x, wfc_ref[...], preferred_element_type=jnp.float32)
    z_ref[...] = z
    zsb_ref[...] = jnp.dot(z, wa_ref[...], preferred_element_type=jnp.float32) + b_ref[...]
    zd_ref[...] = jnp.dot(z, wc_ref[...], preferred_element_type=jnp.float32)


def _node_proj(nfeats, wfc_t, wa_t, wc_t, b2):
    n, din = nfeats.shape
    dout = wfc_t.shape[1]
    de = wa_t.shape[1]
    rb = 1000
    grid = (n // rb,)
    return pl.pallas_call(
        _node_proj_body,
        grid=grid,
        in_specs=[
            pl.BlockSpec((rb, din), lambda i: (i, 0)),
            pl.BlockSpec((din, dout), lambda i: (0, 0)),
            pl.BlockSpec((dout, de), lambda i: (0, 0)),
            pl.BlockSpec((dout, de), lambda i: (0, 0)),
            pl.BlockSpec((1, de), lambda i: (0, 0)),
        ],
        out_specs=[
            pl.BlockSpec((rb, dout), lambda i: (i, 0)),
            pl.BlockSpec((rb, de), lambda i: (i, 0)),
            pl.BlockSpec((rb, de), lambda i: (i, 0)),
        ],
        out_shape=[
            jax.ShapeDtypeStruct((n, dout), jnp.float32),
            jax.ShapeDtypeStruct((n, de), jnp.float32),
            jax.ShapeDtypeStruct((n, de), jnp.float32),
        ],
    )(nfeats, wfc_t, wa_t, wc_t, b2)


def _edge_proj_body(e_ref, wb_ref, ep_ref):
    ep_ref[...] = jnp.dot(e_ref[...], wb_ref[...], preferred_element_type=jnp.float32)


def _edge_proj(efeats, wb_t):
    e, din_e = efeats.shape
    de = wb_t.shape[1]
    rb = 2560
    return pl.pallas_call(
        _edge_proj_body,
        grid=(e // rb,),
        in_specs=[
            pl.BlockSpec((rb, din_e), lambda i: (i, 0)),
            pl.BlockSpec((din_e, de), lambda i: (0, 0)),
        ],
        out_specs=pl.BlockSpec((rb, de), lambda i: (i, 0)),
        out_shape=jax.ShapeDtypeStruct((e, de), jnp.float32),
    )(efeats, wb_t)


def _feat_sc(zsb, zd, ep, src, dst):
    e = src.shape[0]
    de = zsb.shape[1]
    ew = e // NW
    cb = 80
    nchunks = ew // cb
    mesh = plsc.VectorSubcoreMesh(core_axis_name="c", subcore_axis_name="s", num_cores=NC, num_subcores=NS)

    @functools.partial(
        pl.kernel,
        mesh=mesh,
        compiler_params=pltpu.CompilerParams(use_tc_tiling_on_sc=False, needs_layout_passes=False),
        out_type=jax.ShapeDtypeStruct((e, de), jnp.float32),
        scratch_types=[
            pltpu.VMEM((cb,), jnp.int32),
            pltpu.VMEM((cb,), jnp.int32),
            pltpu.VMEM((cb, de), jnp.float32),
            pltpu.VMEM((cb, de), jnp.float32),
            pltpu.VMEM((cb, de), jnp.float32),
            pltpu.VMEM((cb, de), jnp.float32),
            pltpu.SemaphoreType.DMA,
            pltpu.SemaphoreType.DMA,
        ],
    )
    def body(zsb_hbm, zd_hbm, ep_hbm, src_hbm, dst_hbm, out_hbm,
             srcv, dstv, zsr, zdr, epv, fv, sem1, sem2):
        wid = lax.axis_index("s") * NC + lax.axis_index("c")
        base = wid * ew

        def chunk(ci, carry):
            off = base + ci * cb
            pltpu.sync_copy(src_hbm.at[pl.ds(off, cb)], srcv)
            pltpu.sync_copy(dst_hbm.at[pl.ds(off, cb)], dstv)
            cp1 = pltpu.async_copy(zsb_hbm.at[srcv], zsr, sem1)
            cp2 = pltpu.async_copy(zd_hbm.at[dstv], zdr, sem2)
            pltpu.sync_copy(ep_hbm.at[pl.ds(off, cb), :], epv)
            cp1.wait()
            cp2.wait()

            def edge(i, c):
                r = zsr[i, :] + zdr[i, :] + epv[i, :]
                fv[i, :] = _leaky(r)
                return c

            lax.fori_loop(0, cb, edge, 0)
            pltpu.sync_copy(fv, out_hbm.at[pl.ds(off, cb), :])
            return carry

        lax.fori_loop(0, nchunks, chunk, 0)

    return body(zsb, zd, ep, src, dst)


def _attn_body(f_ref, w_ref, a_ref):
    t = jnp.sum(f_ref[...] * w_ref[...], axis=1)
    a_ref[...] = _leaky(_leaky(t))


def _attn_tc(feat, w_coef):
    e, de = feat.shape
    rb = 2048
    grid = (e + rb - 1) // rb
    return pl.pallas_call(
        _attn_body,
        grid=(grid,),
        in_specs=[
            pl.BlockSpec((rb, de), lambda i: (i, 0)),
            pl.BlockSpec((1, de), lambda i: (0, 0)),
        ],
        out_specs=pl.BlockSpec((rb,), lambda i: (i,)),
        out_shape=jax.ShapeDtypeStruct((e,), jnp.float32),
    )(feat, w_coef)


def _rot_gather(x, idx):
    return lax.gather(
        x,
        idx[:, None],
        lax.GatherDimensionNumbers(
            offset_dims=(), collapsed_slice_dims=(0,), start_index_map=(0,)),
        (1,),
        mode=lax.GatherScatterMode.PROMISE_IN_BOUNDS,
    )


def _segmax_sc(attn, dst, n):
    e = attn.shape[0]
    ew = e // NW
    cb = 80
    nchunks = ew // cb
    mesh = plsc.VectorSubcoreMesh(core_axis_name="c", subcore_axis_name="s", num_cores=NC, num_subcores=NS)

    @functools.partial(
        pl.kernel,
        mesh=mesh,
        compiler_params=pltpu.CompilerParams(use_tc_tiling_on_sc=False, needs_layout_passes=False),
        out_type=jax.ShapeDtypeStruct((NW, n), jnp.float32),
        scratch_types=[
            pltpu.VMEM((n,), jnp.float32),
            pltpu.VMEM((cb,), jnp.int32),
            pltpu.VMEM((cb,), jnp.float32),
        ],
    )
    def body(attn_hbm, dst_hbm, out_hbm, tbl, dstv, attnv):
        wid = lax.axis_index("s") * NC + lax.axis_index("c")
        base = wid * ew
        neg = jnp.full((LANES,), _NEG_BIG, jnp.float32)

        def init(i, c):
            tbl[pl.ds(i * LANES, LANES)] = neg
            return c

        lax.fori_loop(0, n // LANES, init, 0)

        iota = lax.iota(jnp.int32, LANES)

        def chunk(ci, carry):
            off = base + ci * cb
            pltpu.sync_copy(dst_hbm.at[pl.ds(off, cb)], dstv)
            pltpu.sync_copy(attn_hbm.at[pl.ds(off, cb)], attnv)

            def grp(g, c):
                k = dstv[pl.ds(g * LANES, LANES)]
                x = attnv[pl.ds(g * LANES, LANES)]

                def rot(r, xx):
                    idx = (iota + r) & (LANES - 1)
                    kr = _rot_gather(k, idx)
                    xr = _rot_gather(xx, idx)
                    return jnp.where(k == kr, jnp.maximum(xx, xr), xx)

                x = lax.fori_loop(1, LANES, rot, x)
                old = plsc.load_gather(tbl, [k])
                plsc.store_scatter(tbl, [k], jnp.maximum(old, x))
                return c

            lax.fori_loop(0, cb // LANES, grp, 0)
            return carry

        lax.fori_loop(0, nchunks, chunk, 0)
        pltpu.sync_copy(tbl, out_hbm.at[wid])

    return body(attn, dst)


def _amax_body(p_ref, o_ref):
    o_ref[...] = jnp.max(p_ref[...], axis=0)


def _amax_tc(part):
    nw, n = part.shape
    return pl.pallas_call(
        _amax_body,
        out_shape=jax.ShapeDtypeStruct((n,), jnp.float32),
    )(part)


def _agg_sc(attn, amax, src, dst, z, zeros_init):
    e = attn.shape[0]
    n, d = z.shape
    dpad = d + LANES
    ew = e // NW
    cb = 80
    nchunks = ew // cb
    rows_per_tile = n // NS
    mesh = plsc.VectorSubcoreMesh(core_axis_name="c", subcore_axis_name="s", num_cores=NC, num_subcores=NS)

    @functools.partial(
        pl.kernel,
        mesh=mesh,
        compiler_params=pltpu.CompilerParams(use_tc_tiling_on_sc=False, needs_layout_passes=False),
        out_type=jax.ShapeDtypeStruct((NC, n, dpad), jnp.float32),
        scratch_types=[
            pltpu.VMEM((n,), jnp.float32),
            pltpu.VMEM((cb,), jnp.int32),
            pltpu.VMEM((cb,), jnp.int32),
            pltpu.VMEM((cb,), jnp.float32),
            pltpu.VMEM((cb,), jnp.float32),
            pltpu.VMEM((cb, d), jnp.float32),
            pltpu.VMEM((cb, dpad), jnp.float32),
            pltpu.VMEM_SHARED((n, dpad), jnp.float32),
            pltpu.SemaphoreType.DMA,
        ],
    )
    def body(attn_hbm, amax_hbm, src_hbm, dst_hbm, z_hbm, zero_hbm, out_hbm,
             amaxv, srcv, dstv, attnv, exv, zrows, scaled, acc, semz):
        cid = lax.axis_index("c")
        sid = lax.axis_index("s")
        wid = sid * NC + cid
        base = wid * ew

        r0 = sid * rows_per_tile
        pltpu.sync_copy(zero_hbm.at[pl.ds(r0, rows_per_tile), :],
                        acc.at[pl.ds(r0, rows_per_tile), :])
        pltpu.sync_copy(amax_hbm, amaxv)
        plsc.subcore_barrier()

        iota = lax.iota(jnp.int32, LANES)

        def chunk(ci, carry):
            off = base + ci * cb
            pltpu.sync_copy(src_hbm.at[pl.ds(off, cb)], srcv)
            pltpu.sync_copy(dst_hbm.at[pl.ds(off, cb)], dstv)
            pltpu.sync_copy(attn_hbm.at[pl.ds(off, cb)], attnv)
            cpz = pltpu.async_copy(z_hbm.at[srcv], zrows, semz)

            def grp(g, c):
                k = dstv[pl.ds(g * LANES, LANES)]
                a = attnv[pl.ds(g * LANES, LANES)]
                am = plsc.load_gather(amaxv, [k])
                exv[pl.ds(g * LANES, LANES)] = jnp.exp(a - am)
                return c

            lax.fori_loop(0, cb // LANES, grp, 0)
            cpz.wait()

            def egrp(g, c):
                ex16 = exv[pl.ds(g * LANES, LANES)]
                for l in range(LANES):
                    i = g * LANES + l
                    s = ex16[l]
                    for j in range(d // LANES):
                        scaled[i, pl.ds(j * LANES, LANES)] = (
                            zrows[i, pl.ds(j * LANES, LANES)] * s)
                    scaled[i, pl.ds(d, LANES)] = jnp.where(
                        iota == 0, s, jnp.zeros((LANES,), jnp.float32))
                return c

            lax.fori_loop(0, cb // LANES, egrp, 0)
            pltpu.sync_copy(scaled, acc.at[dstv], add=True)
            return carry

        lax.fori_loop(0, nchunks, chunk, 0)
        plsc.subcore_barrier()
        pltpu.sync_copy(acc.at[pl.ds(r0, rows_per_tile), :],
                        out_hbm.at[cid, pl.ds(r0, rows_per_tile), :])

    return body(attn, amax, src, dst, z, zeros_init)


def _norm_body(hp_ref, h_ref):
    hs = hp_ref[0] + hp_ref[1]
    d = hs[:, 128:129]
    h_ref[...] = jnp.where(d > 0, hs[:, :128] / d, 0.0)


def _norm_tc(hpart):
    nc, n, dpad = hpart.shape
    d = 128
    rb = 1000
    return pl.pallas_call(
        _norm_body,
        grid=(n // rb,),
        in_specs=[pl.BlockSpec((nc, rb, dpad), lambda i: (0, i, 0))],
        out_specs=pl.BlockSpec((rb, d), lambda i: (i, 0)),
        out_shape=jax.ShapeDtypeStruct((n, d), jnp.float32),
    )(hpart)


def kernel(nfeats, efeats, edge_index, W_fc, W_edge, b_edge, W_coef):
    n, din_n = nfeats.shape
    e, din_e = efeats.shape
    dout = W_fc.shape[0]
    de = W_edge.shape[0]

    src = edge_index[0].astype(jnp.int32)
    dst = edge_index[1].astype(jnp.int32)

    wfc_t = W_fc.T
    wa_t = W_edge[:, :dout].T
    wb_t = W_edge[:, dout:dout + din_e].T
    wc_t = W_edge[:, dout + din_e:].T
    b2 = b_edge.reshape(1, de)

    z, zsb, zd = _node_proj(nfeats, wfc_t, wa_t, wc_t, b2)
    ep = _edge_proj(efeats, wb_t)
    feat = _feat_sc(zsb, zd, ep, src, dst)
    attn = _attn_tc(feat, W_coef)
    part = _segmax_sc(attn, dst, n)
    amax = _amax_tc(part)
    zeros_init = jnp.zeros((n, dout + LANES), jnp.float32)
    hpart = _agg_sc(attn, amax, src, dst, z, zeros_init)
    h = _norm_tc(hpart)
    return h, feat


# speedup vs baseline: 4.4805x; 1.2172x over previous
import functools

import jax
import jax.numpy as jnp
from jax import lax
from jax.experimental import pallas as pl
from jax.experimental.pallas import tpu as pltpu
from jax.experimental.pallas import tpu_sc as plsc

NC = 2
NS = 16
NW = NC * NS
LANES = 16

_NEG_BIG = -3.0e38


def _leaky(x):
    return jnp.where(x >= 0, x, 0.01 * x)


def _node_proj_body(x_ref, wfc_ref, wa_ref, wc_ref, b_ref, z_ref, zsb_ref, zd_ref):
    x = x_ref[...]
    z = jnp.dot(x, wfc_ref[...], preferred_element_type=jnp.float32)
    z_ref[...] = z
    zsb_ref[...] = jnp.dot(z, wa_ref[...], preferred_element_type=jnp.float32) + b_ref[...]
    zd_ref[...] = jnp.dot(z, wc_ref[...], preferred_element_type=jnp.float32)


def _node_proj(nfeats, wfc_t, wa_t, wc_t, b2):
    n, din = nfeats.shape
    dout = wfc_t.shape[1]
    de = wa_t.shape[1]
    rb = 1000
    grid = (n // rb,)
    return pl.pallas_call(
        _node_proj_body,
        grid=grid,
        in_specs=[
            pl.BlockSpec((rb, din), lambda i: (i, 0)),
            pl.BlockSpec((din, dout), lambda i: (0, 0)),
            pl.BlockSpec((dout, de), lambda i: (0, 0)),
            pl.BlockSpec((dout, de), lambda i: (0, 0)),
            pl.BlockSpec((1, de), lambda i: (0, 0)),
        ],
        out_specs=[
            pl.BlockSpec((rb, dout), lambda i: (i, 0)),
            pl.BlockSpec((rb, de), lambda i: (i, 0)),
            pl.BlockSpec((rb, de), lambda i: (i, 0)),
        ],
        out_shape=[
            jax.ShapeDtypeStruct((n, dout), jnp.float32),
            jax.ShapeDtypeStruct((n, de), jnp.float32),
            jax.ShapeDtypeStruct((n, de), jnp.float32),
        ],
    )(nfeats, wfc_t, wa_t, wc_t, b2)


def _edge_proj_body(e_ref, wb_ref, ep_ref):
    ep_ref[...] = jnp.dot(e_ref[...], wb_ref[...], preferred_element_type=jnp.float32)


def _edge_proj(efeats, wb_t):
    e, din_e = efeats.shape
    de = wb_t.shape[1]
    rb = 2560
    return pl.pallas_call(
        _edge_proj_body,
        grid=(e // rb,),
        in_specs=[
            pl.BlockSpec((rb, din_e), lambda i: (i, 0)),
            pl.BlockSpec((din_e, de), lambda i: (0, 0)),
        ],
        out_specs=pl.BlockSpec((rb, de), lambda i: (i, 0)),
        out_shape=jax.ShapeDtypeStruct((e, de), jnp.float32),
    )(efeats, wb_t)


def _feat_sc(zsb, zd, ep, src, dst):
    e = src.shape[0]
    de = zsb.shape[1]
    ew = e // NW
    cb = 80
    nchunks = ew // cb
    mesh = plsc.VectorSubcoreMesh(core_axis_name="c", subcore_axis_name="s", num_cores=NC, num_subcores=NS)

    @functools.partial(
        pl.kernel,
        mesh=mesh,
        compiler_params=pltpu.CompilerParams(use_tc_tiling_on_sc=False, needs_layout_passes=False),
        out_type=jax.ShapeDtypeStruct((e, de), jnp.float32),
        scratch_types=[
            pltpu.VMEM((4, cb), jnp.int32),
            pltpu.VMEM((4, cb), jnp.int32),
            pltpu.VMEM((2, cb, de), jnp.float32),
            pltpu.VMEM((2, cb, de), jnp.float32),
            pltpu.VMEM((2, cb, de), jnp.float32),
            pltpu.VMEM((2, cb, de), jnp.float32),
            pltpu.SemaphoreType.DMA((4,)),
            pltpu.SemaphoreType.DMA((2,)),
            pltpu.SemaphoreType.DMA((2,)),
        ],
    )
    def body(zsb_hbm, zd_hbm, ep_hbm, src_hbm, dst_hbm, out_hbm,
             srcv, dstv, zsr, zdr, epv, fv, semi, semg, semo):
        wid = lax.axis_index("s") * NC + lax.axis_index("c")
        base = wid * ew

        def fire_idx(ci):
            b4 = lax.rem(ci, 4)
            off = base + ci * cb
            pltpu.async_copy(src_hbm.at[pl.ds(off, cb)], srcv.at[b4], semi.at[b4])
            pltpu.async_copy(dst_hbm.at[pl.ds(off, cb)], dstv.at[b4], semi.at[b4])

        def wait_idx(b4):
            pltpu.make_async_copy(src_hbm.at[pl.ds(0, cb)], srcv.at[b4], semi.at[b4]).wait()
            pltpu.make_async_copy(dst_hbm.at[pl.ds(0, cb)], dstv.at[b4], semi.at[b4]).wait()

        def fire_rows(ci):
            b4 = lax.rem(ci, 4)
            b2 = lax.rem(ci, 2)
            off = base + ci * cb
            pltpu.async_copy(zsb_hbm.at[srcv.at[b4]], zsr.at[b2], semg.at[b2])
            pltpu.async_copy(zd_hbm.at[dstv.at[b4]], zdr.at[b2], semg.at[b2])
            pltpu.async_copy(ep_hbm.at[pl.ds(off, cb), :], epv.at[b2], semg.at[b2])

        def wait_rows(b2):
            pltpu.make_async_copy(zsb_hbm.at[srcv.at[0]], zsr.at[b2], semg.at[b2]).wait()
            pltpu.make_async_copy(zd_hbm.at[dstv.at[0]], zdr.at[b2], semg.at[b2]).wait()
            pltpu.make_async_copy(ep_hbm.at[pl.ds(0, cb), :], epv.at[b2], semg.at[b2]).wait()

        def wait_out(b2):
            pltpu.make_async_copy(fv.at[b2], out_hbm.at[pl.ds(0, cb), :], semo.at[b2]).wait()

        fire_idx(0)
        fire_idx(1)
        wait_idx(0)
        fire_rows(0)

        def step(ci, carry):
            b2 = lax.rem(ci, 2)

            @pl.when(ci + 1 < nchunks)
            def _():
                wait_idx(lax.rem(ci + 1, 4))
                fire_rows(ci + 1)

            wait_rows(b2)

            @pl.when(ci >= 2)
            def _():
                wait_out(b2)

            def edge(i, c):
                r = zsr[b2, i, :] + zdr[b2, i, :] + epv[b2, i, :]
                fv[b2, i, :] = _leaky(r)
                return c

            lax.fori_loop(0, cb, edge, 0)
            off = base + ci * cb
            pltpu.async_copy(fv.at[b2], out_hbm.at[pl.ds(off, cb), :], semo.at[b2])

            @pl.when(ci + 2 < nchunks)
            def _():
                fire_idx(ci + 2)

            return carry

        lax.fori_loop(0, nchunks, step, 0)
        wait_out(0)
        wait_out(1)

    return body(zsb, zd, ep, src, dst)


def _attn_body(f_ref, w_ref, a_ref):
    t = jnp.sum(f_ref[...] * w_ref[...], axis=1)
    a_ref[...] = _leaky(_leaky(t))


def _attn_tc(feat, w_coef):
    e, de = feat.shape
    rb = 2048
    grid = (e + rb - 1) // rb
    return pl.pallas_call(
        _attn_body,
        grid=(grid,),
        in_specs=[
            pl.BlockSpec((rb, de), lambda i: (i, 0)),
            pl.BlockSpec((1, de), lambda i: (0, 0)),
        ],
        out_specs=pl.BlockSpec((rb,), lambda i: (i,)),
        out_shape=jax.ShapeDtypeStruct((e,), jnp.float32),
    )(feat, w_coef)


def _rot_gather(x, idx):
    return lax.gather(
        x,
        idx[:, None],
        lax.GatherDimensionNumbers(
            offset_dims=(), collapsed_slice_dims=(0,), start_index_map=(0,)),
        (1,),
        mode=lax.GatherScatterMode.PROMISE_IN_BOUNDS,
    )


def _segmax_sc(attn, dst, n):
    e = attn.shape[0]
    ew = e // NW
    cb = 80
    nchunks = ew // cb
    mesh = plsc.VectorSubcoreMesh(core_axis_name="c", subcore_axis_name="s", num_cores=NC, num_subcores=NS)

    @functools.partial(
        pl.kernel,
        mesh=mesh,
        compiler_params=pltpu.CompilerParams(use_tc_tiling_on_sc=False, needs_layout_passes=False),
        out_type=jax.ShapeDtypeStruct((NW, n), jnp.float32),
        scratch_types=[
            pltpu.VMEM((n,), jnp.float32),
            pltpu.VMEM((cb,), jnp.int32),
            pltpu.VMEM((cb,), jnp.float32),
        ],
    )
    def body(attn_hbm, dst_hbm, out_hbm, tbl, dstv, attnv):
        wid = lax.axis_index("s") * NC + lax.axis_index("c")
        base = wid * ew
        neg = jnp.full((LANES,), _NEG_BIG, jnp.float32)

        def init(i, c):
            tbl[pl.ds(i * LANES, LANES)] = neg
            return c

        lax.fori_loop(0, n // LANES, init, 0)

        iota = lax.iota(jnp.int32, LANES)

        def chunk(ci, carry):
            off = base + ci * cb
            pltpu.sync_copy(dst_hbm.at[pl.ds(off, cb)], dstv)
            pltpu.sync_copy(attn_hbm.at[pl.ds(off, cb)], attnv)

            def grp(g, c):
                k = dstv[pl.ds(g * LANES, LANES)]
                x = attnv[pl.ds(g * LANES, LANES)]

                def rot(r, xx):
                    idx = (iota + r) & (LANES - 1)
                    kr = _rot_gather(k, idx)
                    xr = _rot_gather(xx, idx)
                    return jnp.where(k == kr, jnp.maximum(xx, xr), xx)

                x = lax.fori_loop(1, LANES, rot, x)
                old = plsc.load_gather(tbl, [k])
                plsc.store_scatter(tbl, [k], jnp.maximum(old, x))
                return c

            lax.fori_loop(0, cb // LANES, grp, 0)
            return carry

        lax.fori_loop(0, nchunks, chunk, 0)
        pltpu.sync_copy(tbl, out_hbm.at[wid])

    return body(attn, dst)


def _amax_body(p_ref, o_ref):
    o_ref[...] = jnp.max(p_ref[...], axis=0)


def _amax_tc(part):
    nw, n = part.shape
    return pl.pallas_call(
        _amax_body,
        out_shape=jax.ShapeDtypeStruct((n,), jnp.float32),
    )(part)


def _exp_sc(attn, amax, dst, n):
    e = attn.shape[0]
    ew = e // NW
    cb = 80
    nchunks = ew // cb
    mesh = plsc.VectorSubcoreMesh(core_axis_name="c", subcore_axis_name="s", num_cores=NC, num_subcores=NS)

    @functools.partial(
        pl.kernel,
        mesh=mesh,
        compiler_params=pltpu.CompilerParams(use_tc_tiling_on_sc=False, needs_layout_passes=False),
        out_type=jax.ShapeDtypeStruct((e,), jnp.float32),
        scratch_types=[
            pltpu.VMEM((n,), jnp.float32),
            pltpu.VMEM((4, cb), jnp.int32),
            pltpu.VMEM((4, cb), jnp.float32),
            pltpu.VMEM((2, cb), jnp.float32),
            pltpu.SemaphoreType.DMA((4,)),
            pltpu.SemaphoreType.DMA((2,)),
        ],
    )
    def body(attn_hbm, amax_hbm, dst_hbm, out_hbm, amaxv, dstv, attnv, exv, semi, semo):
        wid = lax.axis_index("s") * NC + lax.axis_index("c")
        base = wid * ew
        pltpu.sync_copy(amax_hbm, amaxv)

        def fire_idx(ci):
            b4 = lax.rem(ci, 4)
            off = base + ci * cb
            pltpu.async_copy(dst_hbm.at[pl.ds(off, cb)], dstv.at[b4], semi.at[b4])
            pltpu.async_copy(attn_hbm.at[pl.ds(off, cb)], attnv.at[b4], semi.at[b4])

        def wait_idx(b4):
            pltpu.make_async_copy(dst_hbm.at[pl.ds(0, cb)], dstv.at[b4], semi.at[b4]).wait()
            pltpu.make_async_copy(attn_hbm.at[pl.ds(0, cb)], attnv.at[b4], semi.at[b4]).wait()

        def wait_out(b2):
            pltpu.make_async_copy(exv.at[b2], out_hbm.at[pl.ds(0, cb)], semo.at[b2]).wait()

        fire_idx(0)
        fire_idx(1)

        def step(ci, carry):
            b4 = lax.rem(ci, 4)
            b2 = lax.rem(ci, 2)
            wait_idx(b4)

            @pl.when(ci >= 2)
            def _():
                wait_out(b2)

            def grp(g, c):
                k = dstv[b4, pl.ds(g * LANES, LANES)]
                a = attnv[b4, pl.ds(g * LANES, LANES)]
                am = plsc.load_gather(amaxv, [k])
                exv[b2, pl.ds(g * LANES, LANES)] = jnp.exp(a - am)
                return c

            lax.fori_loop(0, cb // LANES, grp, 0)
            off = base + ci * cb
            pltpu.async_copy(exv.at[b2], out_hbm.at[pl.ds(off, cb)], semo.at[b2])

            @pl.when(ci + 2 < nchunks)
            def _():
                fire_idx(ci + 2)

            return carry

        lax.fori_loop(0, nchunks, step, 0)
        wait_out(0)
        wait_out(1)

    return body(attn, amax, dst)


def _agg_sc(ex, src, dst, z, zeros_init):
    e = ex.shape[0]
    n, d = z.shape
    dpad = d + 8
    ew = e // NW
    cb = 80
    nchunks = ew // cb
    rows_per_tile = n // NS
    mesh = plsc.VectorSubcoreMesh(core_axis_name="c", subcore_axis_name="s", num_cores=NC, num_subcores=NS)

    @functools.partial(
        pl.kernel,
        mesh=mesh,
        compiler_params=pltpu.CompilerParams(use_tc_tiling_on_sc=False, needs_layout_passes=False),
        out_type=jax.ShapeDtypeStruct((NC, n, dpad), jnp.float32),
        scratch_types=[
            pltpu.VMEM((4, cb), jnp.int32),
            pltpu.VMEM((4, cb), jnp.int32),
            pltpu.VMEM((4, cb), jnp.float32),
            pltpu.VMEM((2, cb, d), jnp.float32),
            pltpu.VMEM((2, cb, dpad), jnp.float32),
            pltpu.VMEM_SHARED((n, dpad), jnp.float32),
            pltpu.SemaphoreType.DMA((4,)),
            pltpu.SemaphoreType.DMA((2,)),
            pltpu.SemaphoreType.DMA((2,)),
        ],
    )
    def body(ex_hbm, src_hbm, dst_hbm, z_hbm, zero_hbm, out_hbm,
             srcv, dstv, exvr, zrows, scaled, acc, semi, semz, semsc):
        cid = lax.axis_index("c")
        sid = lax.axis_index("s")
        wid = sid * NC + cid
        base = wid * ew

        r0 = sid * rows_per_tile
        pltpu.sync_copy(zero_hbm.at[pl.ds(r0, rows_per_tile), :],
                        acc.at[pl.ds(r0, rows_per_tile), :])
        plsc.subcore_barrier()

        iota = lax.iota(jnp.int32, LANES)
        zero16 = jnp.zeros((LANES,), jnp.float32)

        def fire_idx(ci):
            b4 = lax.rem(ci, 4)
            off = base + ci * cb
            pltpu.async_copy(src_hbm.at[pl.ds(off, cb)], srcv.at[b4], semi.at[b4])
            pltpu.async_copy(dst_hbm.at[pl.ds(off, cb)], dstv.at[b4], semi.at[b4])
            pltpu.async_copy(ex_hbm.at[pl.ds(off, cb)], exvr.at[b4], semi.at[b4])

        def wait_idx(b4):
            pltpu.make_async_copy(src_hbm.at[pl.ds(0, cb)], srcv.at[b4], semi.at[b4]).wait()
            pltpu.make_async_copy(dst_hbm.at[pl.ds(0, cb)], dstv.at[b4], semi.at[b4]).wait()
            pltpu.make_async_copy(ex_hbm.at[pl.ds(0, cb)], exvr.at[b4], semi.at[b4]).wait()

        def fire_rows(ci):
            b4 = lax.rem(ci, 4)
            b2 = lax.rem(ci, 2)
            pltpu.async_copy(z_hbm.at[srcv.at[b4]], zrows.at[b2], semz.at[b2])

        def wait_rows(b2):
            pltpu.make_async_copy(z_hbm.at[srcv.at[0]], zrows.at[b2], semz.at[b2]).wait()

        def wait_scat(b2):
            pltpu.make_async_copy(scaled.at[b2], acc.at[dstv.at[0]], semsc.at[b2]).wait()

        fire_idx(0)
        fire_idx(1)
        wait_idx(0)
        fire_rows(0)

        def step(ci, carry):
            b4 = lax.rem(ci, 4)
            b2 = lax.rem(ci, 2)

            @pl.when(ci + 1 < nchunks)
            def _():
                wait_idx(lax.rem(ci + 1, 4))
                fire_rows(ci + 1)

            wait_rows(b2)

            @pl.when(ci >= 2)
            def _():
                wait_scat(b2)

            def egrp(g, c):
                ex16 = exvr[b4, pl.ds(g * LANES, LANES)]
                for l in range(LANES):
                    i = g * LANES + l
                    s = ex16[l]
                    for j in range(d // LANES):
                        t = zrows[b2, i, pl.ds(j * LANES, LANES)] * s
                        scaled[b2, i, pl.ds(j * LANES, LANES)] = t
                        if j == d // LANES - 1:
                            thi = _rot_gather(t, (iota + 8) & (LANES - 1))
                            scaled[b2, i, pl.ds(d - 8, LANES)] = jnp.where(
                                iota < 8, thi, jnp.where(iota == 8, s, zero16))
                return c

            lax.fori_loop(0, cb // LANES, egrp, 0)
            pltpu.async_copy(scaled.at[b2], acc.at[dstv.at[b4]], semsc.at[b2], add=True)

            @pl.when(ci + 2 < nchunks)
            def _():
                fire_idx(ci + 2)

            return carry

        lax.fori_loop(0, nchunks, step, 0)
        wait_scat(0)
        wait_scat(1)
        plsc.subcore_barrier()
        pltpu.sync_copy(acc.at[pl.ds(r0, rows_per_tile), :],
                        out_hbm.at[cid, pl.ds(r0, rows_per_tile), :])

    return body(ex, src, dst, z, zeros_init)


def _norm_body(hp_ref, h_ref):
    hs = hp_ref[0] + hp_ref[1]
    d = hs[:, 128:129]
    h_ref[...] = jnp.where(d > 0, hs[:, :128] / d, 0.0)


def _norm_tc(hpart):
    nc, n, dpad = hpart.shape
    d = 128
    rb = 1000
    return pl.pallas_call(
        _norm_body,
        grid=(n // rb,),
        in_specs=[pl.BlockSpec((nc, rb, dpad), lambda i: (0, i, 0))],
        out_specs=pl.BlockSpec((rb, d), lambda i: (i, 0)),
        out_shape=jax.ShapeDtypeStruct((n, d), jnp.float32),
    )(hpart)


def kernel(nfeats, efeats, edge_index, W_fc, W_edge, b_edge, W_coef):
    n, din_n = nfeats.shape
    e, din_e = efeats.shape
    dout = W_fc.shape[0]
    de = W_edge.shape[0]

    src = edge_index[0].astype(jnp.int32)
    dst = edge_index[1].astype(jnp.int32)

    wfc_t = W_fc.T
    wa_t = W_edge[:, :dout].T
    wb_t = W_edge[:, dout:dout + din_e].T
    wc_t = W_edge[:, dout + din_e:].T
    b2 = b_edge.reshape(1, de)

    z, zsb, zd = _node_proj(nfeats, wfc_t, wa_t, wc_t, b2)
    ep = _edge_proj(efeats, wb_t)
    feat = _feat_sc(zsb, zd, ep, src, dst)
    attn = _attn_tc(feat, W_coef)
    part = _segmax_sc(attn, dst, n)
    amax = _amax_tc(part)
    ex = _exp_sc(attn, amax, dst, n)
    zeros_init = jnp.zeros((n, dout + 8), jnp.float32)
    hpart = _agg_sc(ex, src, dst, z, zeros_init)
    h = _norm_tc(hpart)
    return h, feat


# speedup vs baseline: 5.6986x; 1.2719x over previous
import functools

import jax
import jax.numpy as jnp
from jax import lax
from jax.experimental import pallas as pl
from jax.experimental.pallas import tpu as pltpu
from jax.experimental.pallas import tpu_sc as plsc

NC = 2
NS = 16
NW = NC * NS
LANES = 16

_NEG_BIG = -3.0e38


def _leaky(x):
    return jnp.where(x >= 0, x, 0.01 * x)


def _proj_body(x_ref, e_ref, wfc_ref, wa_ref, wc_ref, wb_ref, b_ref,
               z_ref, zsb_ref, zd_ref, ep_ref):
    x = x_ref[...]
    z = jnp.dot(x, wfc_ref[...], preferred_element_type=jnp.float32)
    z_ref[...] = z
    zsb_ref[...] = jnp.dot(z, wa_ref[...], preferred_element_type=jnp.float32) + b_ref[...]
    zd_ref[...] = jnp.dot(z, wc_ref[...], preferred_element_type=jnp.float32)
    ep_ref[...] = jnp.dot(e_ref[...], wb_ref[...], preferred_element_type=jnp.float32)


def _proj_tc(nfeats, efeats, wfc_t, wa_t, wc_t, wb_t, b2):
    n, din = nfeats.shape
    e, din_e = efeats.shape
    dout = wfc_t.shape[1]
    de = wa_t.shape[1]
    g = 125
    nb = n // g
    eb = e // g
    return pl.pallas_call(
        _proj_body,
        grid=(g,),
        in_specs=[
            pl.BlockSpec((nb, din), lambda i: (i, 0)),
            pl.BlockSpec((eb, din_e), lambda i: (i, 0)),
            pl.BlockSpec((din, dout), lambda i: (0, 0)),
            pl.BlockSpec((dout, de), lambda i: (0, 0)),
            pl.BlockSpec((dout, de), lambda i: (0, 0)),
            pl.BlockSpec((din_e, de), lambda i: (0, 0)),
            pl.BlockSpec((1, de), lambda i: (0, 0)),
        ],
        out_specs=[
            pl.BlockSpec((nb, dout), lambda i: (i, 0)),
            pl.BlockSpec((nb, de), lambda i: (i, 0)),
            pl.BlockSpec((nb, de), lambda i: (i, 0)),
            pl.BlockSpec((eb, de), lambda i: (i, 0)),
        ],
        out_shape=[
            jax.ShapeDtypeStruct((n, dout), jnp.float32),
            jax.ShapeDtypeStruct((n, de), jnp.float32),
            jax.ShapeDtypeStruct((n, de), jnp.float32),
            jax.ShapeDtypeStruct((e, de), jnp.float32),
        ],
    )(nfeats, efeats, wfc_t, wa_t, wc_t, wb_t, b2)


def _rot_gather(x, idx):
    return lax.gather(
        x,
        idx[:, None],
        lax.GatherDimensionNumbers(
            offset_dims=(), collapsed_slice_dims=(0,), start_index_map=(0,)),
        (1,),
        mode=lax.GatherScatterMode.PROMISE_IN_BOUNDS,
    )


def _feat_attn_sc(zsb, zd, ep, src, dst, wvec, n):
    e = src.shape[0]
    de = zsb.shape[1]
    ew = e // NW
    cb = 80
    nchunks = ew // cb
    mesh = plsc.VectorSubcoreMesh(core_axis_name="c", subcore_axis_name="s", num_cores=NC, num_subcores=NS)

    @functools.partial(
        pl.kernel,
        mesh=mesh,
        compiler_params=pltpu.CompilerParams(use_tc_tiling_on_sc=False, needs_layout_passes=False),
        out_type=[
            jax.ShapeDtypeStruct((e, de), jnp.float32),
            jax.ShapeDtypeStruct((e,), jnp.float32),
            jax.ShapeDtypeStruct((NW, n), jnp.float32),
        ],
        scratch_types=[
            pltpu.VMEM((4, cb), jnp.int32),
            pltpu.VMEM((4, cb), jnp.int32),
            pltpu.VMEM((2, cb, de), jnp.float32),
            pltpu.VMEM((2, cb, de), jnp.float32),
            pltpu.VMEM((2, cb, de), jnp.float32),
            pltpu.VMEM((2, cb, de), jnp.float32),
            pltpu.VMEM((2, cb), jnp.float32),
            pltpu.VMEM((16,), jnp.float32),
            pltpu.VMEM((n,), jnp.float32),
            pltpu.SemaphoreType.DMA((4,)),
            pltpu.SemaphoreType.DMA((2,)),
            pltpu.SemaphoreType.DMA((2,)),
        ],
    )
    def body(zsb_hbm, zd_hbm, ep_hbm, src_hbm, dst_hbm, w_hbm,
             out_hbm, attn_hbm, part_hbm,
             srcv, dstv, zsr, zdr, epv, fv, avr, wv_ref, tbl, semi, semg, semo):
        wid = lax.axis_index("s") * NC + lax.axis_index("c")
        base = wid * ew
        pltpu.sync_copy(w_hbm, wv_ref)
        wv = wv_ref[...]
        iota = lax.iota(jnp.int32, LANES)
        neg = jnp.full((LANES,), _NEG_BIG, jnp.float32)

        def init(i, c):
            tbl[pl.ds(i * LANES, LANES)] = neg
            return c

        lax.fori_loop(0, n // LANES, init, 0)

        def fire_idx(ci):
            b4 = lax.rem(ci, 4)
            off = base + ci * cb
            pltpu.async_copy(src_hbm.at[pl.ds(off, cb)], srcv.at[b4], semi.at[b4])
            pltpu.async_copy(dst_hbm.at[pl.ds(off, cb)], dstv.at[b4], semi.at[b4])

        def wait_idx(b4):
            pltpu.make_async_copy(src_hbm.at[pl.ds(0, cb)], srcv.at[b4], semi.at[b4]).wait()
            pltpu.make_async_copy(dst_hbm.at[pl.ds(0, cb)], dstv.at[b4], semi.at[b4]).wait()

        def fire_rows(ci):
            b4 = lax.rem(ci, 4)
            b2 = lax.rem(ci, 2)
            off = base + ci * cb
            pltpu.async_copy(zsb_hbm.at[srcv.at[b4]], zsr.at[b2], semg.at[b2])
            pltpu.async_copy(zd_hbm.at[dstv.at[b4]], zdr.at[b2], semg.at[b2])
            pltpu.async_copy(ep_hbm.at[pl.ds(off, cb), :], epv.at[b2], semg.at[b2])

        def wait_rows(b2):
            pltpu.make_async_copy(zsb_hbm.at[srcv.at[0]], zsr.at[b2], semg.at[b2]).wait()
            pltpu.make_async_copy(zd_hbm.at[srcv.at[0]], zdr.at[b2], semg.at[b2]).wait()
            pltpu.make_async_copy(ep_hbm.at[pl.ds(0, cb), :], epv.at[b2], semg.at[b2]).wait()

        def wait_out(b2):
            pltpu.make_async_copy(fv.at[b2], out_hbm.at[pl.ds(0, cb), :], semo.at[b2]).wait()
            pltpu.make_async_copy(avr.at[b2], attn_hbm.at[pl.ds(0, cb)], semo.at[b2]).wait()

        fire_idx(0)
        fire_idx(1)
        wait_idx(0)
        fire_rows(0)

        def step(ci, carry):
            b4 = lax.rem(ci, 4)
            b2 = lax.rem(ci, 2)

            @pl.when(ci + 1 < nchunks)
            def _():
                wait_idx(lax.rem(ci + 1, 4))
                fire_rows(ci + 1)

            wait_rows(b2)

            @pl.when(ci >= 2)
            def _():
                wait_out(b2)

            def grp(g, c):
                acc = jnp.zeros((LANES,), jnp.float32)
                for l in range(LANES):
                    i = g * LANES + l
                    r = zsr[b2, i, :] + zdr[b2, i, :] + epv[b2, i, :]
                    f = _leaky(r)
                    fv[b2, i, :] = f
                    csum = plsc.cumsum(f * wv)
                    sv = _rot_gather(csum, iota * 0 + (LANES - 1))
                    acc = jnp.where(iota == l, sv, acc)
                attn16 = _leaky(_leaky(acc))
                avr[b2, pl.ds(g * LANES, LANES)] = attn16
                k = dstv[b4, pl.ds(g * LANES, LANES)]
                cnts, _ = plsc.scan_count(k)
                hasdup = jnp.max(cnts) > jnp.min(cnts)

                def dedup(xx):
                    def rot(r_, x_):
                        idx = (iota + r_) & (LANES - 1)
                        kr = _rot_gather(k, idx)
                        xr = _rot_gather(x_, idx)
                        return jnp.where(k == kr, jnp.maximum(x_, xr), x_)

                    return lax.fori_loop(1, LANES, rot, xx)

                x = lax.cond(hasdup, dedup, lambda xx: xx, attn16)
                old = plsc.load_gather(tbl, [k])
                plsc.store_scatter(tbl, [k], jnp.maximum(old, x))
                return c

            lax.fori_loop(0, cb // LANES, grp, 0)
            off = base + ci * cb
            pltpu.async_copy(fv.at[b2], out_hbm.at[pl.ds(off, cb), :], semo.at[b2])
            pltpu.async_copy(avr.at[b2], attn_hbm.at[pl.ds(off, cb)], semo.at[b2])

            @pl.when(ci + 2 < nchunks)
            def _():
                fire_idx(ci + 2)

            return carry

        lax.fori_loop(0, nchunks, step, 0)
        wait_out(0)
        wait_out(1)
        pltpu.sync_copy(tbl, part_hbm.at[wid])

    return body(zsb, zd, ep, src, dst, wvec)


def _exp_sc(attn, part, dst, n):
    e = attn.shape[0]
    ew = e // NW
    cb = 80
    nchunks = ew // cb
    mesh = plsc.VectorSubcoreMesh(core_axis_name="c", subcore_axis_name="s", num_cores=NC, num_subcores=NS)

    @functools.partial(
        pl.kernel,
        mesh=mesh,
        compiler_params=pltpu.CompilerParams(use_tc_tiling_on_sc=False, needs_layout_passes=False),
        out_type=jax.ShapeDtypeStruct((e,), jnp.float32),
        scratch_types=[
            pltpu.VMEM((n,), jnp.float32),
            pltpu.VMEM((2, n), jnp.float32),
            pltpu.VMEM((4, cb), jnp.int32),
            pltpu.VMEM((4, cb), jnp.float32),
            pltpu.VMEM((2, cb), jnp.float32),
            pltpu.SemaphoreType.DMA((4,)),
            pltpu.SemaphoreType.DMA((2,)),
            pltpu.SemaphoreType.DMA((2,)),
        ],
    )
    def body(attn_hbm, part_hbm, dst_hbm, out_hbm, amaxv, ptv, dstv, attnv, exv,
             semi, semo, semp):
        wid = lax.axis_index("s") * NC + lax.axis_index("c")
        base = wid * ew

        pltpu.sync_copy(part_hbm.at[0], amaxv)
        pltpu.async_copy(part_hbm.at[1], ptv.at[1], semp.at[1])

        def tstep(t, carry):
            bt = lax.rem(t, 2)
            pltpu.make_async_copy(part_hbm.at[0], ptv.at[bt], semp.at[bt]).wait()

            @pl.when(t + 1 < NW)
            def _():
                pltpu.async_copy(part_hbm.at[t + 1], ptv.at[lax.rem(t + 1, 2)],
                                 semp.at[lax.rem(t + 1, 2)])

            def red(i, c):
                sl = pl.ds(i * LANES, LANES)
                amaxv[sl] = jnp.maximum(amaxv[sl], ptv[bt, sl])
                return c

            lax.fori_loop(0, n // LANES, red, 0)
            return carry

        lax.fori_loop(1, NW, tstep, 0)

        def fire_idx(ci):
            b4 = lax.rem(ci, 4)
            off = base + ci * cb
            pltpu.async_copy(dst_hbm.at[pl.ds(off, cb)], dstv.at[b4], semi.at[b4])
            pltpu.async_copy(attn_hbm.at[pl.ds(off, cb)], attnv.at[b4], semi.at[b4])

        def wait_idx(b4):
            pltpu.make_async_copy(dst_hbm.at[pl.ds(0, cb)], dstv.at[b4], semi.at[b4]).wait()
            pltpu.make_async_copy(attn_hbm.at[pl.ds(0, cb)], attnv.at[b4], semi.at[b4]).wait()

        def wait_out(b2):
            pltpu.make_async_copy(exv.at[b2], out_hbm.at[pl.ds(0, cb)], semo.at[b2]).wait()

        fire_idx(0)
        fire_idx(1)

        def step(ci, carry):
            b4 = lax.rem(ci, 4)
            b2 = lax.rem(ci, 2)
            wait_idx(b4)

            @pl.when(ci >= 2)
            def _():
                wait_out(b2)

            def grp(g, c):
                k = dstv[b4, pl.ds(g * LANES, LANES)]
                a = attnv[b4, pl.ds(g * LANES, LANES)]
                am = plsc.load_gather(amaxv, [k])
                exv[b2, pl.ds(g * LANES, LANES)] = jnp.exp(a - am)
                return c

            lax.fori_loop(0, cb // LANES, grp, 0)
            off = base + ci * cb
            pltpu.async_copy(exv.at[b2], out_hbm.at[pl.ds(off, cb)], semo.at[b2])

            @pl.when(ci + 2 < nchunks)
            def _():
                fire_idx(ci + 2)

            return carry

        lax.fori_loop(0, nchunks, step, 0)
        wait_out(0)
        wait_out(1)

    return body(attn, part, dst)


def _agg_sc(ex, src, dst, z):
    e = ex.shape[0]
    n, d = z.shape
    dpad = d + 8
    ew = e // NW
    cb = 80
    nchunks = ew // cb
    rows_per_tile = n // NS
    mesh = plsc.VectorSubcoreMesh(core_axis_name="c", subcore_axis_name="s", num_cores=NC, num_subcores=NS)

    @functools.partial(
        pl.kernel,
        mesh=mesh,
        compiler_params=pltpu.CompilerParams(use_tc_tiling_on_sc=False, needs_layout_passes=False),
        out_type=jax.ShapeDtypeStruct((NC, n, dpad), jnp.float32),
        scratch_types=[
            pltpu.VMEM((4, cb), jnp.int32),
            pltpu.VMEM((4, cb), jnp.int32),
            pltpu.VMEM((4, cb), jnp.float32),
            pltpu.VMEM((2, cb, d), jnp.float32),
            pltpu.VMEM((2, cb, dpad), jnp.float32),
            pltpu.VMEM_SHARED((n, dpad), jnp.float32),
            pltpu.SemaphoreType.DMA((4,)),
            pltpu.SemaphoreType.DMA((2,)),
            pltpu.SemaphoreType.DMA((2,)),
        ],
    )
    def body(ex_hbm, src_hbm, dst_hbm, z_hbm, out_hbm,
             srcv, dstv, exvr, zrows, scaled, acc, semi, semz, semsc):
        cid = lax.axis_index("c")
        sid = lax.axis_index("s")
        wid = sid * NC + cid
        base = wid * ew
        iota = lax.iota(jnp.int32, LANES)
        zero16 = jnp.zeros((LANES,), jnp.float32)

        def zero_row(i, c):
            for j in range(dpad // LANES):
                scaled[0, i, pl.ds(j * LANES, LANES)] = zero16
            scaled[0, i, pl.ds(dpad - LANES, LANES)] = zero16
            scaled[1, i, pl.ds(dpad - LANES, LANES)] = zero16
            return c

        lax.fori_loop(0, cb, zero_row, 0)
        r0 = sid * rows_per_tile
        nfull = rows_per_tile // cb
        rrem = rows_per_tile - nfull * cb

        def zcp(i, c):
            pltpu.sync_copy(scaled.at[0],
                            acc.at[pl.ds(r0 + i * cb, cb), :])
            return c

        lax.fori_loop(0, nfull, zcp, 0)
        if rrem:
            pltpu.sync_copy(scaled.at[0, pl.ds(0, rrem), :],
                            acc.at[pl.ds(r0 + nfull * cb, rrem), :])
        plsc.subcore_barrier()

        def fire_idx(ci):
            b4 = lax.rem(ci, 4)
            off = base + ci * cb
            pltpu.async_copy(src_hbm.at[pl.ds(off, cb)], srcv.at[b4], semi.at[b4])
            pltpu.async_copy(dst_hbm.at[pl.ds(off, cb)], dstv.at[b4], semi.at[b4])
            pltpu.async_copy(ex_hbm.at[pl.ds(off, cb)], exvr.at[b4], semi.at[b4])

        def wait_idx(b4):
            pltpu.make_async_copy(src_hbm.at[pl.ds(0, cb)], srcv.at[b4], semi.at[b4]).wait()
            pltpu.make_async_copy(dst_hbm.at[pl.ds(0, cb)], dstv.at[b4], semi.at[b4]).wait()
            pltpu.make_async_copy(ex_hbm.at[pl.ds(0, cb)], exvr.at[b4], semi.at[b4]).wait()

        def fire_rows(ci):
            b4 = lax.rem(ci, 4)
            b2 = lax.rem(ci, 2)
            pltpu.async_copy(z_hbm.at[srcv.at[b4]], zrows.at[b2], semz.at[b2])

        def wait_rows(b2):
            pltpu.make_async_copy(z_hbm.at[srcv.at[0]], zrows.at[b2], semz.at[b2]).wait()

        def wait_scat(b2):
            pltpu.make_async_copy(scaled.at[b2], acc.at[dstv.at[0]], semsc.at[b2]).wait()

        fire_idx(0)
        fire_idx(1)
        wait_idx(0)
        fire_rows(0)

        def step(ci, carry):
            b4 = lax.rem(ci, 4)
            b2 = lax.rem(ci, 2)

            @pl.when(ci + 1 < nchunks)
            def _():
                wait_idx(lax.rem(ci + 1, 4))
                fire_rows(ci + 1)

            wait_rows(b2)

            @pl.when(ci >= 2)
            def _():
                wait_scat(b2)

            b2s = iota * 0 + b2

            def egrp(g, c):
                ex16 = exvr[b4, pl.ds(g * LANES, LANES)]
                for l in range(LANES):
                    i = g * LANES + l
                    sv = _rot_gather(ex16, iota * 0 + l)
                    for j in range(d // LANES):
                        scaled[b2, i, pl.ds(j * LANES, LANES)] = (
                            zrows[b2, i, pl.ds(j * LANES, LANES)] * sv)
                rows = g * LANES + iota
                plsc.store_scatter(scaled, [b2s, rows, iota * 0 + d], ex16)
                return c

            lax.fori_loop(0, cb // LANES, egrp, 0)
            pltpu.async_copy(scaled.at[b2], acc.at[dstv.at[b4]], semsc.at[b2], add=True)

            @pl.when(ci + 2 < nchunks)
            def _():
                fire_idx(ci + 2)

            return carry

        lax.fori_loop(0, nchunks, step, 0)
        wait_scat(0)
        wait_scat(1)
        plsc.subcore_barrier()
        pltpu.sync_copy(acc.at[pl.ds(r0, rows_per_tile), :],
                        out_hbm.at[cid, pl.ds(r0, rows_per_tile), :])

    return body(ex, src, dst, z)


def _norm_body(hp_ref, h_ref):
    hs = hp_ref[0] + hp_ref[1]
    d = hs[:, 128:129]
    h_ref[...] = jnp.where(d > 0, hs[:, :128] / d, 0.0)


def _norm_tc(hpart):
    nc, n, dpad = hpart.shape
    d = 128
    rb = 1000
    return pl.pallas_call(
        _norm_body,
        grid=(n // rb,),
        in_specs=[pl.BlockSpec((nc, rb, dpad), lambda i: (0, i, 0))],
        out_specs=pl.BlockSpec((rb, d), lambda i: (i, 0)),
        out_shape=jax.ShapeDtypeStruct((n, d), jnp.float32),
    )(hpart)


def kernel(nfeats, efeats, edge_index, W_fc, W_edge, b_edge, W_coef):
    n, din_n = nfeats.shape
    e, din_e = efeats.shape
    dout = W_fc.shape[0]
    de = W_edge.shape[0]

    src = edge_index[0].astype(jnp.int32)
    dst = edge_index[1].astype(jnp.int32)

    wfc_t = W_fc.T
    wa_t = W_edge[:, :dout].T
    wb_t = W_edge[:, dout:dout + din_e].T
    wc_t = W_edge[:, dout + din_e:].T
    b2 = b_edge.reshape(1, de)

    z, zsb, zd, ep = _proj_tc(nfeats, efeats, wfc_t, wa_t, wc_t, wb_t, b2)
    feat, attn, part = _feat_attn_sc(zsb, zd, ep, src, dst, W_coef.reshape(de), n)
    ex = _exp_sc(attn, part, dst, n)
    hpart = _agg_sc(ex, src, dst, z)
    h = _norm_tc(hpart)
    return h, feat


# speedup vs baseline: 6.1683x; 1.0824x over previous
import functools

import jax
import jax.numpy as jnp
from jax import lax
from jax.experimental import pallas as pl
from jax.experimental.pallas import tpu as pltpu
from jax.experimental.pallas import tpu_sc as plsc

NC = 2
NS = 16
NW = NC * NS
LANES = 16

_NEG_BIG = -3.0e38


def _leaky(x):
    return jnp.where(x >= 0, x, 0.01 * x)


def _proj_body(x_ref, e_ref, wfc_ref, wa_ref, wc_ref, wb_ref, b_ref,
               z_ref, zsb_ref, zd_ref, ep_ref):
    x = x_ref[...]
    z = jnp.dot(x, wfc_ref[...], preferred_element_type=jnp.float32)
    z_ref[...] = z
    zsb_ref[...] = jnp.dot(z, wa_ref[...], preferred_element_type=jnp.float32) + b_ref[...]
    zd_ref[...] = jnp.dot(z, wc_ref[...], preferred_element_type=jnp.float32)
    ep_ref[...] = jnp.dot(e_ref[...], wb_ref[...], preferred_element_type=jnp.float32)


def _proj_tc(nfeats, efeats, wfc_t, wa_t, wc_t, wb_t, b2):
    n, din = nfeats.shape
    e, din_e = efeats.shape
    dout = wfc_t.shape[1]
    de = wa_t.shape[1]
    g = 125
    nb = n // g
    eb = e // g
    return pl.pallas_call(
        _proj_body,
        grid=(g,),
        in_specs=[
            pl.BlockSpec((nb, din), lambda i: (i, 0)),
            pl.BlockSpec((eb, din_e), lambda i: (i, 0)),
            pl.BlockSpec((din, dout), lambda i: (0, 0)),
            pl.BlockSpec((dout, de), lambda i: (0, 0)),
            pl.BlockSpec((dout, de), lambda i: (0, 0)),
            pl.BlockSpec((din_e, de), lambda i: (0, 0)),
            pl.BlockSpec((1, de), lambda i: (0, 0)),
        ],
        out_specs=[
            pl.BlockSpec((nb, dout), lambda i: (i, 0)),
            pl.BlockSpec((nb, de), lambda i: (i, 0)),
            pl.BlockSpec((nb, de), lambda i: (i, 0)),
            pl.BlockSpec((eb, de), lambda i: (i, 0)),
        ],
        out_shape=[
            jax.ShapeDtypeStruct((n, dout), jnp.float32),
            jax.ShapeDtypeStruct((n, de), jnp.float32),
            jax.ShapeDtypeStruct((n, de), jnp.float32),
            jax.ShapeDtypeStruct((e, de), jnp.float32),
        ],
    )(nfeats, efeats, wfc_t, wa_t, wc_t, wb_t, b2)


def _rot_gather(x, idx):
    return lax.gather(
        x,
        idx[:, None],
        lax.GatherDimensionNumbers(
            offset_dims=(), collapsed_slice_dims=(0,), start_index_map=(0,)),
        (1,),
        mode=lax.GatherScatterMode.PROMISE_IN_BOUNDS,
    )


def _feat_attn_sc(zsb, zd, ep, src, dst, wvec, n):
    e = src.shape[0]
    de = zsb.shape[1]
    ew = e // NW
    cb = 80
    nchunks = ew // cb
    mesh = plsc.VectorSubcoreMesh(core_axis_name="c", subcore_axis_name="s", num_cores=NC, num_subcores=NS)

    @functools.partial(
        pl.kernel,
        mesh=mesh,
        compiler_params=pltpu.CompilerParams(use_tc_tiling_on_sc=False, needs_layout_passes=False),
        out_type=[
            jax.ShapeDtypeStruct((e, de), jnp.float32),
            jax.ShapeDtypeStruct((e,), jnp.float32),
            jax.ShapeDtypeStruct((NC, n), jnp.float32),
        ],
        scratch_types=[
            pltpu.VMEM((4, cb), jnp.int32),
            pltpu.VMEM((4, cb), jnp.int32),
            pltpu.VMEM((2, cb, de), jnp.float32),
            pltpu.VMEM((2, cb, de), jnp.float32),
            pltpu.VMEM((2, cb, de), jnp.float32),
            pltpu.VMEM((2, cb, de), jnp.float32),
            pltpu.VMEM((2, cb), jnp.float32),
            pltpu.VMEM((16,), jnp.float32),
            pltpu.VMEM((n,), jnp.float32),
            pltpu.VMEM((640,), jnp.float32),
            pltpu.VMEM((640,), jnp.float32),
            pltpu.VMEM_SHARED((NS, n), jnp.float32),
            pltpu.SemaphoreType.DMA((4,)),
            pltpu.SemaphoreType.DMA((2,)),
            pltpu.SemaphoreType.DMA((2,)),
        ],
    )
    def body(zsb_hbm, zd_hbm, ep_hbm, src_hbm, dst_hbm, w_hbm,
             out_hbm, attn_hbm, part_hbm,
             srcv, dstv, zsr, zdr, epv, fv, avr, wv_ref, tbl,
             redb, stgb, shr, semi, semg, semo):
        cid = lax.axis_index("c")
        sid = lax.axis_index("s")
        wid = sid * NC + cid
        base = wid * ew
        nslc = 640
        pltpu.sync_copy(w_hbm, wv_ref)
        wv = wv_ref[...]
        iota = lax.iota(jnp.int32, LANES)
        neg = jnp.full((LANES,), _NEG_BIG, jnp.float32)

        def init(i, c):
            tbl[pl.ds(i * LANES, LANES)] = neg
            return c

        lax.fori_loop(0, n // LANES, init, 0)

        def fire_idx(ci):
            b4 = lax.rem(ci, 4)
            off = base + ci * cb
            pltpu.async_copy(src_hbm.at[pl.ds(off, cb)], srcv.at[b4], semi.at[b4])
            pltpu.async_copy(dst_hbm.at[pl.ds(off, cb)], dstv.at[b4], semi.at[b4])

        def wait_idx(b4):
            pltpu.make_async_copy(src_hbm.at[pl.ds(0, cb)], srcv.at[b4], semi.at[b4]).wait()
            pltpu.make_async_copy(dst_hbm.at[pl.ds(0, cb)], dstv.at[b4], semi.at[b4]).wait()

        def fire_rows(ci):
            b4 = lax.rem(ci, 4)
            b2 = lax.rem(ci, 2)
            off = base + ci * cb
            pltpu.async_copy(zsb_hbm.at[srcv.at[b4]], zsr.at[b2], semg.at[b2])
            pltpu.async_copy(zd_hbm.at[dstv.at[b4]], zdr.at[b2], semg.at[b2])
            pltpu.async_copy(ep_hbm.at[pl.ds(off, cb), :], epv.at[b2], semg.at[b2])

        def wait_rows(b2):
            pltpu.make_async_copy(zsb_hbm.at[srcv.at[0]], zsr.at[b2], semg.at[b2]).wait()
            pltpu.make_async_copy(zd_hbm.at[srcv.at[0]], zdr.at[b2], semg.at[b2]).wait()
            pltpu.make_async_copy(ep_hbm.at[pl.ds(0, cb), :], epv.at[b2], semg.at[b2]).wait()

        def wait_out(b2):
            pltpu.make_async_copy(fv.at[b2], out_hbm.at[pl.ds(0, cb), :], semo.at[b2]).wait()
            pltpu.make_async_copy(avr.at[b2], attn_hbm.at[pl.ds(0, cb)], semo.at[b2]).wait()

        fire_idx(0)
        fire_idx(1)
        wait_idx(0)
        fire_rows(0)

        def step(ci, carry):
            b4 = lax.rem(ci, 4)
            b2 = lax.rem(ci, 2)

            @pl.when(ci + 1 < nchunks)
            def _():
                wait_idx(lax.rem(ci + 1, 4))
                fire_rows(ci + 1)

            wait_rows(b2)

            @pl.when(ci >= 2)
            def _():
                wait_out(b2)

            def grp(g, c):
                acc = jnp.zeros((LANES,), jnp.float32)
                for l in range(LANES):
                    i = g * LANES + l
                    r = zsr[b2, i, :] + zdr[b2, i, :] + epv[b2, i, :]
                    f = _leaky(r)
                    fv[b2, i, :] = f
                    csum = plsc.cumsum(f * wv)
                    sv = _rot_gather(csum, iota * 0 + (LANES - 1))
                    acc = jnp.where(iota == l, sv, acc)
                attn16 = _leaky(_leaky(acc))
                avr[b2, pl.ds(g * LANES, LANES)] = attn16
                k = dstv[b4, pl.ds(g * LANES, LANES)]
                cnts, _ = plsc.scan_count(k)
                hasdup = jnp.max(cnts) > jnp.min(cnts)

                def dedup(xx):
                    def rot(r_, x_):
                        idx = (iota + r_) & (LANES - 1)
                        kr = _rot_gather(k, idx)
                        xr = _rot_gather(x_, idx)
                        return jnp.where(k == kr, jnp.maximum(x_, xr), x_)

                    return lax.fori_loop(1, LANES, rot, xx)

                x = lax.cond(hasdup, dedup, lambda xx: xx, attn16)
                old = plsc.load_gather(tbl, [k])
                plsc.store_scatter(tbl, [k], jnp.maximum(old, x))
                return c

            lax.fori_loop(0, cb // LANES, grp, 0)
            off = base + ci * cb
            pltpu.async_copy(fv.at[b2], out_hbm.at[pl.ds(off, cb), :], semo.at[b2])
            pltpu.async_copy(avr.at[b2], attn_hbm.at[pl.ds(off, cb)], semo.at[b2])

            @pl.when(ci + 2 < nchunks)
            def _():
                fire_idx(ci + 2)

            return carry

        lax.fori_loop(0, nchunks, step, 0)
        wait_out(0)
        wait_out(1)

        pltpu.sync_copy(tbl, shr.at[sid])
        plsc.subcore_barrier()

        def reduce_slice(o0, olen):
            pltpu.sync_copy(shr.at[0, pl.ds(o0, olen)], redb.at[pl.ds(0, olen)])

            def tred(t, c):
                pltpu.sync_copy(shr.at[t, pl.ds(o0, olen)], stgb.at[pl.ds(0, olen)])

                def mx(i, cc):
                    sl = pl.ds(i * LANES, LANES)
                    redb[sl] = jnp.maximum(redb[sl], stgb[sl])
                    return cc

                lax.fori_loop(0, olen // LANES, mx, 0)
                return c

            lax.fori_loop(1, NS, tred, 0)
            pltpu.sync_copy(redb.at[pl.ds(0, olen)],
                            part_hbm.at[cid, pl.ds(o0, olen)])

        @pl.when(sid < NS - 1)
        def _():
            reduce_slice(sid * nslc, nslc)

        @pl.when(sid == NS - 1)
        def _():
            reduce_slice((NS - 1) * nslc, n - (NS - 1) * nslc)

    return body(zsb, zd, ep, src, dst, wvec)


def _exp_sc(attn, part, dst, n):
    e = attn.shape[0]
    ew = e // NW
    cb = 80
    nchunks = ew // cb
    mesh = plsc.VectorSubcoreMesh(core_axis_name="c", subcore_axis_name="s", num_cores=NC, num_subcores=NS)

    @functools.partial(
        pl.kernel,
        mesh=mesh,
        compiler_params=pltpu.CompilerParams(use_tc_tiling_on_sc=False, needs_layout_passes=False),
        out_type=jax.ShapeDtypeStruct((e,), jnp.float32),
        scratch_types=[
            pltpu.VMEM((n,), jnp.float32),
            pltpu.VMEM((n,), jnp.float32),
            pltpu.VMEM((4, cb), jnp.int32),
            pltpu.VMEM((4, cb), jnp.float32),
            pltpu.VMEM((2, cb), jnp.float32),
            pltpu.SemaphoreType.DMA((4,)),
            pltpu.SemaphoreType.DMA((2,)),
        ],
    )
    def body(attn_hbm, part_hbm, dst_hbm, out_hbm, amaxv, ptv, dstv, attnv, exv,
             semi, semo):
        wid = lax.axis_index("s") * NC + lax.axis_index("c")
        base = wid * ew

        pltpu.sync_copy(part_hbm.at[0], amaxv)
        pltpu.sync_copy(part_hbm.at[1], ptv)

        def red(i, c):
            sl = pl.ds(i * LANES, LANES)
            amaxv[sl] = jnp.maximum(amaxv[sl], ptv[sl])
            return c

        lax.fori_loop(0, n // LANES, red, 0)

        def fire_idx(ci):
            b4 = lax.rem(ci, 4)
            off = base + ci * cb
            pltpu.async_copy(dst_hbm.at[pl.ds(off, cb)], dstv.at[b4], semi.at[b4])
            pltpu.async_copy(attn_hbm.at[pl.ds(off, cb)], attnv.at[b4], semi.at[b4])

        def wait_idx(b4):
            pltpu.make_async_copy(dst_hbm.at[pl.ds(0, cb)], dstv.at[b4], semi.at[b4]).wait()
            pltpu.make_async_copy(attn_hbm.at[pl.ds(0, cb)], attnv.at[b4], semi.at[b4]).wait()

        def wait_out(b2):
            pltpu.make_async_copy(exv.at[b2], out_hbm.at[pl.ds(0, cb)], semo.at[b2]).wait()

        fire_idx(0)
        fire_idx(1)

        def step(ci, carry):
            b4 = lax.rem(ci, 4)
            b2 = lax.rem(ci, 2)
            wait_idx(b4)

            @pl.when(ci >= 2)
            def _():
                wait_out(b2)

            def grp(g, c):
                k = dstv[b4, pl.ds(g * LANES, LANES)]
                a = attnv[b4, pl.ds(g * LANES, LANES)]
                am = plsc.load_gather(amaxv, [k])
                exv[b2, pl.ds(g * LANES, LANES)] = jnp.exp(a - am)
                return c

            lax.fori_loop(0, cb // LANES, grp, 0)
            off = base + ci * cb
            pltpu.async_copy(exv.at[b2], out_hbm.at[pl.ds(off, cb)], semo.at[b2])

            @pl.when(ci + 2 < nchunks)
            def _():
                fire_idx(ci + 2)

            return carry

        lax.fori_loop(0, nchunks, step, 0)
        wait_out(0)
        wait_out(1)

    return body(attn, part, dst)


def _agg_sc(ex, src, dst, z):
    e = ex.shape[0]
    n, d = z.shape
    dpad = d + 8
    ew = e // NW
    cb = 80
    nchunks = ew // cb
    rows_per_tile = n // NS
    mesh = plsc.VectorSubcoreMesh(core_axis_name="c", subcore_axis_name="s", num_cores=NC, num_subcores=NS)

    @functools.partial(
        pl.kernel,
        mesh=mesh,
        compiler_params=pltpu.CompilerParams(use_tc_tiling_on_sc=False, needs_layout_passes=False),
        out_type=jax.ShapeDtypeStruct((NC, n, dpad), jnp.float32),
        scratch_types=[
            pltpu.VMEM((4, cb), jnp.int32),
            pltpu.VMEM((4, cb), jnp.int32),
            pltpu.VMEM((4, cb), jnp.float32),
            pltpu.VMEM((2, cb, d), jnp.float32),
            pltpu.VMEM((2, cb, dpad), jnp.float32),
            pltpu.VMEM_SHARED((n, dpad), jnp.float32),
            pltpu.SemaphoreType.DMA((4,)),
            pltpu.SemaphoreType.DMA((2,)),
            pltpu.SemaphoreType.DMA((2,)),
        ],
    )
    def body(ex_hbm, src_hbm, dst_hbm, z_hbm, out_hbm,
             srcv, dstv, exvr, zrows, scaled, acc, semi, semz, semsc):
        cid = lax.axis_index("c")
        sid = lax.axis_index("s")
        wid = sid * NC + cid
        base = wid * ew
        iota = lax.iota(jnp.int32, LANES)
        zero16 = jnp.zeros((LANES,), jnp.float32)

        def zero_row(i, c):
            for j in range(dpad // LANES):
                scaled[0, i, pl.ds(j * LANES, LANES)] = zero16
            scaled[0, i, pl.ds(dpad - LANES, LANES)] = zero16
            scaled[1, i, pl.ds(dpad - LANES, LANES)] = zero16
            return c

        lax.fori_loop(0, cb, zero_row, 0)
        r0 = sid * rows_per_tile
        nfull = rows_per_tile // cb
        rrem = rows_per_tile - nfull * cb

        def zcp(i, c):
            pltpu.sync_copy(scaled.at[0],
                            acc.at[pl.ds(r0 + i * cb, cb), :])
            return c

        lax.fori_loop(0, nfull, zcp, 0)
        if rrem:
            pltpu.sync_copy(scaled.at[0, pl.ds(0, rrem), :],
                            acc.at[pl.ds(r0 + nfull * cb, rrem), :])
        plsc.subcore_barrier()

        def fire_idx(ci):
            b4 = lax.rem(ci, 4)
            off = base + ci * cb
            pltpu.async_copy(src_hbm.at[pl.ds(off, cb)], srcv.at[b4], semi.at[b4])
            pltpu.async_copy(dst_hbm.at[pl.ds(off, cb)], dstv.at[b4], semi.at[b4])
            pltpu.async_copy(ex_hbm.at[pl.ds(off, cb)], exvr.at[b4], semi.at[b4])

        def wait_idx(b4):
            pltpu.make_async_copy(src_hbm.at[pl.ds(0, cb)], srcv.at[b4], semi.at[b4]).wait()
            pltpu.make_async_copy(dst_hbm.at[pl.ds(0, cb)], dstv.at[b4], semi.at[b4]).wait()
            pltpu.make_async_copy(ex_hbm.at[pl.ds(0, cb)], exvr.at[b4], semi.at[b4]).wait()

        def fire_rows(ci):
            b4 = lax.rem(ci, 4)
            b2 = lax.rem(ci, 2)
            hf = cb // 2
            pltpu.async_copy(z_hbm.at[srcv.at[b4, pl.ds(0, hf)]],
                             zrows.at[b2, pl.ds(0, hf)], semz.at[b2])
            pltpu.async_copy(z_hbm.at[srcv.at[b4, pl.ds(hf, hf)]],
                             zrows.at[b2, pl.ds(hf, hf)], semz.at[b2])

        def wait_rows(b2):
            pltpu.make_async_copy(z_hbm.at[srcv.at[0]], zrows.at[b2], semz.at[b2]).wait()

        def wait_scat(b2):
            pltpu.make_async_copy(scaled.at[b2], acc.at[dstv.at[0]], semsc.at[b2]).wait()

        fire_idx(0)
        fire_idx(1)
        wait_idx(0)
        fire_rows(0)

        def step(ci, carry):
            b4 = lax.rem(ci, 4)
            b2 = lax.rem(ci, 2)

            @pl.when(ci + 1 < nchunks)
            def _():
                wait_idx(lax.rem(ci + 1, 4))
                fire_rows(ci + 1)

            wait_rows(b2)

            @pl.when(ci >= 2)
            def _():
                wait_scat(b2)

            b2s = iota * 0 + b2

            def egrp(g, c):
                ex16 = exvr[b4, pl.ds(g * LANES, LANES)]
                for l in range(LANES):
                    i = g * LANES + l
                    sv = _rot_gather(ex16, iota * 0 + l)
                    for j in range(d // LANES):
                        scaled[b2, i, pl.ds(j * LANES, LANES)] = (
                            zrows[b2, i, pl.ds(j * LANES, LANES)] * sv)
                rows = g * LANES + iota
                plsc.store_scatter(scaled, [b2s, rows, iota * 0 + d], ex16)
                return c

            lax.fori_loop(0, cb // LANES, egrp, 0)
            pltpu.async_copy(scaled.at[b2], acc.at[dstv.at[b4]], semsc.at[b2], add=True)

            @pl.when(ci + 2 < nchunks)
            def _():
                fire_idx(ci + 2)

            return carry

        lax.fori_loop(0, nchunks, step, 0)
        wait_scat(0)
        wait_scat(1)
        plsc.subcore_barrier()
        pltpu.sync_copy(acc.at[pl.ds(r0, rows_per_tile), :],
                        out_hbm.at[cid, pl.ds(r0, rows_per_tile), :])

    return body(ex, src, dst, z)


def _norm_body(hp_ref, h_ref):
    hs = hp_ref[0] + hp_ref[1]
    d = hs[:, 128:129]
    h_ref[...] = jnp.where(d > 0, hs[:, :128] / d, 0.0)


def _norm_tc(hpart):
    nc, n, dpad = hpart.shape
    d = 128
    rb = 1000
    return pl.pallas_call(
        _norm_body,
        grid=(n // rb,),
        in_specs=[pl.BlockSpec((nc, rb, dpad), lambda i: (0, i, 0))],
        out_specs=pl.BlockSpec((rb, d), lambda i: (i, 0)),
        out_shape=jax.ShapeDtypeStruct((n, d), jnp.float32),
    )(hpart)


def kernel(nfeats, efeats, edge_index, W_fc, W_edge, b_edge, W_coef):
    n, din_n = nfeats.shape
    e, din_e = efeats.shape
    dout = W_fc.shape[0]
    de = W_edge.shape[0]

    src = edge_index[0].astype(jnp.int32)
    dst = edge_index[1].astype(jnp.int32)

    wfc_t = W_fc.T
    wa_t = W_edge[:, :dout].T
    wb_t = W_edge[:, dout:dout + din_e].T
    wc_t = W_edge[:, dout + din_e:].T
    b2 = b_edge.reshape(1, de)

    z, zsb, zd, ep = _proj_tc(nfeats, efeats, wfc_t, wa_t, wc_t, wb_t, b2)
    feat, attn, part = _feat_attn_sc(zsb, zd, ep, src, dst, W_coef.reshape(de), n)
    ex = _exp_sc(attn, part, dst, n)
    hpart = _agg_sc(ex, src, dst, z)
    h = _norm_tc(hpart)
    return h, feat
